# baseline (speedup 1.0000x reference)
import functools

import jax
import jax.numpy as jnp
from jax import lax
from jax.experimental import pallas as pl
from jax.experimental.pallas import tpu as pltpu
from jax.experimental.pallas import tpu_sc as plsc

_N = 10000
_E = 320000
_D = 128
_NC = 2
_NS = 16
_NW = _NC * _NS
_EPW = _E // _NW
_L = 16

_mesh = plsc.VectorSubcoreMesh(core_axis_name="c", subcore_axis_name="s")


def _worker_id():
    return lax.axis_index("s") * _NC + lax.axis_index("c")


_UNROLL = 25


def _zero_vmem(ref, n):
    zeros = jnp.zeros((_L,), jnp.float32)
    step = _UNROLL * _L

    def body(i, carry):
        for j in range(_UNROLL):
            ref[pl.ds(i * step + j * _L, _L)] = zeros
        return carry

    lax.fori_loop(0, n // step, body, 0)


_BATCH = 25


@functools.partial(
    pl.kernel,
    out_type=jax.ShapeDtypeStruct((_NW, _N), jnp.float32),
    mesh=_mesh,
    scratch_types=[
        pltpu.VMEM((_EPW,), jnp.int32),
        pltpu.VMEM((_N,), jnp.float32),
        pltpu.SemaphoreType.DMA,
    ],
    compiler_params=pltpu.CompilerParams(needs_layout_passes=False, use_tc_tiling_on_sc=False),
)
def _sc_degree(ei_hbm, out_hbm, idx_v, acc_v, sem):
    wid = _worker_id()
    cp = pltpu.async_copy(ei_hbm.at[1, pl.ds(wid * _EPW, _EPW)], idx_v, sem)
    _zero_vmem(acc_v, _N)
    cp.wait()
    ones = jnp.ones((_L,), jnp.float32)
    step = _UNROLL * _L

    def body(i, carry):
        for j0 in range(0, _UNROLL, _BATCH):
            idxs = [idx_v[pl.ds(i * step + (j0 + k) * _L, _L)]
                    for k in range(_BATCH)]
            for k in range(_BATCH):
                plsc.addupdate_scatter(acc_v, [idxs[k]], ones)
        return carry

    lax.fori_loop(0, _EPW // step, body, 0)
    pltpu.sync_copy(acc_v, out_hbm.at[wid])


@functools.partial(
    pl.kernel,
    out_type=jax.ShapeDtypeStruct((_NW, _N), jnp.float32),
    mesh=_mesh,
    scratch_types=[
        pltpu.VMEM((_N,), jnp.float32),
        pltpu.VMEM((_EPW,), jnp.int32),
        pltpu.VMEM((_EPW,), jnp.int32),
        pltpu.VMEM((_N,), jnp.float32),
        pltpu.SemaphoreType.DMA,
        pltpu.SemaphoreType.DMA,
        pltpu.SemaphoreType.DMA,
    ],
    compiler_params=pltpu.CompilerParams(needs_layout_passes=False, use_tc_tiling_on_sc=False),
)
def _sc_scatter(g_hbm, ei_hbm, out_hbm, g_v, src_v, dst_v, acc_v,
                sem_g, sem_s, sem_d):
    wid = _worker_id()
    base = wid * _EPW
    cp_g = pltpu.async_copy(g_hbm.at[0], g_v, sem_g)
    cp_s = pltpu.async_copy(ei_hbm.at[0, pl.ds(base, _EPW)], src_v, sem_s)
    cp_d = pltpu.async_copy(ei_hbm.at[1, pl.ds(base, _EPW)], dst_v, sem_d)
    _zero_vmem(acc_v, _N)
    cp_g.wait()
    cp_s.wait()
    cp_d.wait()

    step = _UNROLL * _L

    def body(i, carry):
        for j0 in range(0, _UNROLL, _BATCH):
            ss = [src_v[pl.ds(i * step + (j0 + k) * _L, _L)]
                  for k in range(_BATCH)]
            vals = [plsc.load_gather(g_v, [s]) for s in ss]
            dd = [dst_v[pl.ds(i * step + (j0 + k) * _L, _L)]
                  for k in range(_BATCH)]
            for k in range(_BATCH):
                plsc.addupdate_scatter(acc_v, [dd[k]], vals[k])
        return carry

    lax.fori_loop(0, _EPW // step, body, 0)
    pltpu.sync_copy(acc_v, out_hbm.at[wid])


def _tc_matvec_body(x_ref, W1_ref, W2_ref, Wo_ref, b1_ref, b2_ref, bo_ref,
                    y0_ref, consts_ref):
    v = jnp.dot(W2_ref[...], Wo_ref[...])
    u = jnp.dot(W1_ref[...], v)
    y0_ref[...] = jnp.dot(x_ref[...], u).T
    beta = jnp.dot(b1_ref[...], v)
    c2 = jnp.dot(b2_ref[...], Wo_ref[...]) + bo_ref[...]
    consts_ref[...] = jnp.concatenate([beta, c2], axis=1)


_tc_matvec = pl.pallas_call(
    _tc_matvec_body,
    out_shape=(
        jax.ShapeDtypeStruct((1, _N), jnp.float32),
        jax.ShapeDtypeStruct((1, 2), jnp.float32),
    ),
)


def _tc_prep0_body(degp_ref, y0_ref, dinv_ref, g0_ref):
    deg = jnp.sum(degp_ref[...], axis=0, keepdims=True) + 1.0
    r = lax.rsqrt(deg)
    r = r * (1.5 - 0.5 * deg * r * r)
    dinv = r * (1.5 - 0.5 * deg * r * r)
    dinv_ref[...] = dinv
    g0_ref[...] = dinv * y0_ref[...]


_tc_prep0 = pl.pallas_call(
    _tc_prep0_body,
    out_shape=(
        jax.ShapeDtypeStruct((1, _N), jnp.float32),
        jax.ShapeDtypeStruct((1, _N), jnp.float32),
    ),
)


def _tc_step_body(sp_ref, g_ref, dinv_ref, consts_ref, gout_ref):
    s = jnp.sum(sp_ref[...], axis=0, keepdims=True)
    y = dinv_ref[...] * (s + g_ref[...]) + consts_ref[0, 0]
    gout_ref[...] = dinv_ref[...] * y


_tc_step = pl.pallas_call(
    _tc_step_body,
    out_shape=jax.ShapeDtypeStruct((1, _N), jnp.float32),
)


def _tc_final_body(sp_ref, g_ref, dinv_ref, consts_ref, out_ref):
    s = jnp.sum(sp_ref[...], axis=0, keepdims=True)
    y = dinv_ref[...] * (s + g_ref[...]) + consts_ref[0, 1]
    out_ref[...] = jnp.clip(y, -4.0, 4.0)


_tc_final = pl.pallas_call(
    _tc_final_body,
    out_shape=jax.ShapeDtypeStruct((1, _N), jnp.float32),
)


def kernel(x, edge_index, W1, b1, W2, b2, W_out, b_out):
    b1r = b1.reshape(1, _D)
    b2r = b2.reshape(1, _D)
    bor = b_out.reshape(1, 1)

    degp = _sc_degree(edge_index)
    y0, consts = _tc_matvec(x, W1, W2, W_out, b1r, b2r, bor)
    dinv, g0 = _tc_prep0(degp, y0)
    s0p = _sc_scatter(g0, edge_index)
    g1 = _tc_step(s0p, g0, dinv, consts)
    s1p = _sc_scatter(g1, edge_index)
    out = _tc_final(s1p, g1, dinv, consts)
    return out.reshape(_N, 1)


# speedup vs baseline: 1.0019x; 1.0019x over previous
import functools

import jax
import jax.numpy as jnp
from jax import lax
from jax.experimental import pallas as pl
from jax.experimental.pallas import tpu as pltpu
from jax.experimental.pallas import tpu_sc as plsc

_N = 10000
_E = 320000
_D = 128
_NC = 2
_NS = 16
_NW = _NC * _NS
_EPW = _E // _NW
_L = 16

_mesh = plsc.VectorSubcoreMesh(core_axis_name="c", subcore_axis_name="s")


def _worker_id():
    return lax.axis_index("s") * _NC + lax.axis_index("c")


_UNROLL = 25


def _zero_vmem(ref, n):
    zeros = jnp.zeros((_L,), jnp.float32)
    step = _UNROLL * _L

    def body(i, carry):
        for j in range(_UNROLL):
            ref[pl.ds(i * step + j * _L, _L)] = zeros
        return carry

    lax.fori_loop(0, n // step, body, 0)


_BATCH = 5


@functools.partial(
    pl.kernel,
    out_type=jax.ShapeDtypeStruct((_NW, _N), jnp.float32),
    mesh=_mesh,
    scratch_types=[
        pltpu.VMEM((_EPW,), jnp.int32),
        pltpu.VMEM((_N,), jnp.float32),
        pltpu.SemaphoreType.DMA,
    ],
    compiler_params=pltpu.CompilerParams(needs_layout_passes=False, use_tc_tiling_on_sc=False),
)
def _sc_degree(ei_hbm, out_hbm, idx_v, acc_v, sem):
    wid = _worker_id()
    cp = pltpu.async_copy(ei_hbm.at[1, pl.ds(wid * _EPW, _EPW)], idx_v, sem)
    _zero_vmem(acc_v, _N)
    cp.wait()
    ones = jnp.ones((_L,), jnp.float32)
    step = _UNROLL * _L

    def body(i, carry):
        for j0 in range(0, _UNROLL, _BATCH):
            idxs = [idx_v[pl.ds(i * step + (j0 + k) * _L, _L)]
                    for k in range(_BATCH)]
            for k in range(_BATCH):
                plsc.addupdate_scatter(acc_v, [idxs[k]], ones)
        return carry

    lax.fori_loop(0, _EPW // step, body, 0)
    pltpu.sync_copy(acc_v, out_hbm.at[wid])


@functools.partial(
    pl.kernel,
    out_type=jax.ShapeDtypeStruct((_NW, _N), jnp.float32),
    mesh=_mesh,
    scratch_types=[
        pltpu.VMEM((_N,), jnp.float32),
        pltpu.VMEM((_EPW,), jnp.int32),
        pltpu.VMEM((_EPW,), jnp.int32),
        pltpu.VMEM((_N,), jnp.float32),
        pltpu.SemaphoreType.DMA,
        pltpu.SemaphoreType.DMA,
        pltpu.SemaphoreType.DMA,
    ],
    compiler_params=pltpu.CompilerParams(needs_layout_passes=False, use_tc_tiling_on_sc=False),
)
def _sc_scatter(g_hbm, ei_hbm, out_hbm, g_v, src_v, dst_v, acc_v,
                sem_g, sem_s, sem_d):
    wid = _worker_id()
    base = wid * _EPW
    cp_g = pltpu.async_copy(g_hbm.at[0], g_v, sem_g)
    cp_s = pltpu.async_copy(ei_hbm.at[0, pl.ds(base, _EPW)], src_v, sem_s)
    cp_d = pltpu.async_copy(ei_hbm.at[1, pl.ds(base, _EPW)], dst_v, sem_d)
    _zero_vmem(acc_v, _N)
    cp_g.wait()
    cp_s.wait()
    cp_d.wait()

    step = _UNROLL * _L

    def body(i, carry):
        for j0 in range(0, _UNROLL, _BATCH):
            ss = [src_v[pl.ds(i * step + (j0 + k) * _L, _L)]
                  for k in range(_BATCH)]
            vals = [plsc.load_gather(g_v, [s]) for s in ss]
            dd = [dst_v[pl.ds(i * step + (j0 + k) * _L, _L)]
                  for k in range(_BATCH)]
            for k in range(_BATCH):
                plsc.addupdate_scatter(acc_v, [dd[k]], vals[k])
        return carry

    lax.fori_loop(0, _EPW // step, body, 0)
    pltpu.sync_copy(acc_v, out_hbm.at[wid])


def _tc_matvec_body(x_ref, W1_ref, W2_ref, Wo_ref, b1_ref, b2_ref, bo_ref,
                    y0_ref, consts_ref):
    v = jnp.dot(W2_ref[...], Wo_ref[...])
    u = jnp.dot(W1_ref[...], v)
    y0_ref[...] = jnp.dot(x_ref[...], u).T
    beta = jnp.dot(b1_ref[...], v)
    c2 = jnp.dot(b2_ref[...], Wo_ref[...]) + bo_ref[...]
    consts_ref[...] = jnp.concatenate([beta, c2], axis=1)


_tc_matvec = pl.pallas_call(
    _tc_matvec_body,
    out_shape=(
        jax.ShapeDtypeStruct((1, _N), jnp.float32),
        jax.ShapeDtypeStruct((1, 2), jnp.float32),
    ),
)


def _tc_prep0_body(degp_ref, y0_ref, dinv_ref, g0_ref):
    deg = jnp.sum(degp_ref[...], axis=0, keepdims=True) + 1.0
    r = lax.rsqrt(deg)
    r = r * (1.5 - 0.5 * deg * r * r)
    dinv = r * (1.5 - 0.5 * deg * r * r)
    dinv_ref[...] = dinv
    g0_ref[...] = dinv * y0_ref[...]


_tc_prep0 = pl.pallas_call(
    _tc_prep0_body,
    out_shape=(
        jax.ShapeDtypeStruct((1, _N), jnp.float32),
        jax.ShapeDtypeStruct((1, _N), jnp.float32),
    ),
)


def _tc_step_body(sp_ref, g_ref, dinv_ref, consts_ref, gout_ref):
    s = jnp.sum(sp_ref[...], axis=0, keepdims=True)
    y = dinv_ref[...] * (s + g_ref[...]) + consts_ref[0, 0]
    gout_ref[...] = dinv_ref[...] * y


_tc_step = pl.pallas_call(
    _tc_step_body,
    out_shape=jax.ShapeDtypeStruct((1, _N), jnp.float32),
)


def _tc_final_body(sp_ref, g_ref, dinv_ref, consts_ref, out_ref):
    s = jnp.sum(sp_ref[...], axis=0, keepdims=True)
    y = dinv_ref[...] * (s + g_ref[...]) + consts_ref[0, 1]
    out_ref[...] = jnp.clip(y, -4.0, 4.0)


_tc_final = pl.pallas_call(
    _tc_final_body,
    out_shape=jax.ShapeDtypeStruct((1, _N), jnp.float32),
)


def kernel(x, edge_index, W1, b1, W2, b2, W_out, b_out):
    b1r = b1.reshape(1, _D)
    b2r = b2.reshape(1, _D)
    bor = b_out.reshape(1, 1)

    degp = _sc_degree(edge_index)
    y0, consts = _tc_matvec(x, W1, W2, W_out, b1r, b2r, bor)
    dinv, g0 = _tc_prep0(degp, y0)
    s0p = _sc_scatter(g0, edge_index)
    g1 = _tc_step(s0p, g0, dinv, consts)
    s1p = _sc_scatter(g1, edge_index)
    out = _tc_final(s1p, g1, dinv, consts)
    return out.reshape(_N, 1)
